# Initial kernel scaffold; baseline (speedup 1.0000x reference)
import jax
import jax.numpy as jnp
from jax.experimental import pallas as pl


def kernel(real_double_features, fake_double_features, real_double_labels, fake_double_labels, real_centroids, fake_centroids, neg_offset):
    raise NotImplementedError("write your pallas kernel here")


# speedup vs baseline: 11.3231x; 11.3231x over previous
import functools

import jax
import jax.numpy as jnp
from jax import lax
from jax.experimental import pallas as pl
from jax.experimental.pallas import tpu as pltpu

_MARGIN = 0.2
_MOMENTUM = 0.9
_NC = 16
_D = 512
_BLK = 2048


def _fused_body(rl0, rl1, fl0, fl1, noff, r_ref, f_ref, rc_ref, fc_ref,
                loss_out, nrc_out, nfc_out,
                rsum, fsum, rcnt, fcnt, lacc):
    i = pl.program_id(0)
    nsteps = pl.num_programs(0)

    @pl.when(i == 0)
    def _init():
        rsum[...] = jnp.zeros_like(rsum)
        fsum[...] = jnp.zeros_like(fsum)
        rcnt[...] = jnp.zeros_like(rcnt)
        fcnt[...] = jnp.zeros_like(fcnt)
        lacc[...] = jnp.zeros_like(lacc)

    f = f_ref[...]
    r = r_ref[...]
    rcid = rl0[0] * 4 + rl1[0]
    fcid = fl0[0] * 4 + fl1[0]
    ncid = lax.rem(fcid + 1 + noff[0], _NC)

    iota_c = lax.broadcasted_iota(jnp.int32, (_NC, _BLK), 0)
    oh_r = (iota_c == rcid).astype(jnp.float32)
    oh_f = (iota_c == fcid).astype(jnp.float32)
    oh_n = (iota_c == ncid).astype(jnp.float32)

    dn = (((1,), (0,)), ((), ()))
    rsum[...] += lax.dot_general(oh_r, r, dn, preferred_element_type=jnp.float32)
    fsum[...] += lax.dot_general(oh_f, f, dn, preferred_element_type=jnp.float32)
    rcnt[...] += jnp.sum(oh_r, axis=1, keepdims=True)
    fcnt[...] += jnp.sum(oh_f, axis=1, keepdims=True)

    cp = rc_ref[...] - 1e-6
    gt = lax.dot_general(cp, f, (((1,), (1,)), ((), ())),
                         preferred_element_type=jnp.float32)
    cn2 = jnp.sum(cp * cp, axis=1, keepdims=True)
    term = cn2 - 2.0 * gt
    posd = jnp.sum(oh_f * term, axis=0, keepdims=True)
    negd = jnp.sum(oh_n * term, axis=0, keepdims=True)
    rown = jnp.reshape(jnp.sum(f * f, axis=1), (1, _BLK))
    dpos = jnp.sqrt(jnp.maximum(rown + posd, 0.0))
    dneg = jnp.sqrt(jnp.maximum(rown + negd, 0.0))
    lacc[...] += jnp.sum(jnp.maximum(dpos - dneg + _MARGIN, 0.0), keepdims=True)

    @pl.when(i == nsteps - 1)
    def _fin():
        n_fake = nsteps * _BLK
        loss_out[...] = lacc[...] / n_fake
        rmean = rsum[...] / jnp.maximum(rcnt[...], 1.0)
        fmean = fsum[...] / jnp.maximum(fcnt[...], 1.0)
        rup = _MOMENTUM * rc_ref[...] + (1.0 - _MOMENTUM) * rmean
        fup = _MOMENTUM * fc_ref[...] + (1.0 - _MOMENTUM) * fmean
        nrc_out[...] = jnp.where(rcnt[...] > 0.0, rup, rc_ref[...])
        nfc_out[...] = jnp.where(fcnt[...] > 0.0, fup, fc_ref[...])


@jax.jit
def _run(r, f, rl0, rl1, fl0, fl1, noff, rc, fc):
    n = r.shape[0]
    grid = n // _BLK
    idx3 = lambda i: (i, 0, 0)
    row3 = pl.BlockSpec((1, 1, _BLK), idx3)
    rows = pl.BlockSpec((_BLK, _D), lambda i: (i, 0))
    full = pl.BlockSpec((_NC, _D), lambda i: (0, 0))
    out = pl.pallas_call(
        _fused_body,
        grid=(grid,),
        in_specs=[row3, row3, row3, row3, row3, rows, rows, full, full],
        out_specs=[pl.BlockSpec((1, 1), lambda i: (0, 0)), full, full],
        out_shape=[
            jax.ShapeDtypeStruct((1, 1), jnp.float32),
            jax.ShapeDtypeStruct((_NC, _D), jnp.float32),
            jax.ShapeDtypeStruct((_NC, _D), jnp.float32),
        ],
        scratch_shapes=[
            pltpu.VMEM((_NC, _D), jnp.float32),
            pltpu.VMEM((_NC, _D), jnp.float32),
            pltpu.VMEM((_NC, 1), jnp.float32),
            pltpu.VMEM((_NC, 1), jnp.float32),
            pltpu.VMEM((1, 1), jnp.float32),
        ],
    )(rl0, rl1, fl0, fl1, noff, r, f, rc, fc)
    return out


def kernel(real_double_features, fake_double_features, real_double_labels,
           fake_double_labels, real_centroids, fake_centroids, neg_offset):
    n = real_double_features.shape[0]
    g = n // _BLK
    shp = (g, 1, _BLK)
    rl0 = real_double_labels[:, 0].reshape(shp)
    rl1 = real_double_labels[:, 1].reshape(shp)
    fl0 = fake_double_labels[:, 0].reshape(shp)
    fl1 = fake_double_labels[:, 1].reshape(shp)
    noff = neg_offset.reshape(shp)
    loss, nrc, nfc = _run(real_double_features, fake_double_features,
                          rl0, rl1, fl0, fl1, noff,
                          real_centroids, fake_centroids)
    return loss.reshape(()), nrc, nfc
